# baseline (speedup 1.0000x reference)
import functools

import jax
import jax.numpy as jnp
import numpy as np
from jax import lax
from jax.experimental import pallas as pl
from jax.experimental.pallas import tpu as pltpu
from jax.experimental.pallas import tpu_sc as plsc

N_ATOMS = 10000
N_BONDS = 320000
MAX_NB = 32
ATOM_FDIM = 128
BOND_FDIM = 144
HIDDEN = 256
DEPTH = 3
N_MOLS = 100

NC, NS, L = 2, 16, 16
NW = NC * NS
NWD = HIDDEN // 2
WCH = NWD // L

_w = np.arange(NWD)
_LO = 32 * (_w // 16) + (_w % 16)
_HI = _LO + 16
Q = np.concatenate([_LO, _HI])

APAD = 10240
BPAD = 327680
A_PER_W = APAD // NW
B_PER_W = BPAD // NW
AB = 4
SGN = A_PER_W // AB
KB = 64
EGN = B_PER_W // KB
_RB = 8
NBUF = 4

_MESH = plsc.VectorSubcoreMesh(
    core_axis_name="c", subcore_axis_name="s", num_cores=NC, num_subcores=NS)

_MASKHI = np.uint32(0xFFFF0000)
_RND = np.uint32(0x8000)


def _up(w):
    u = lax.bitcast_convert_type(w, jnp.uint32)
    lo = lax.bitcast_convert_type(u << np.uint32(16), jnp.float32)
    hi = lax.bitcast_convert_type(u & _MASKHI, jnp.float32)
    return lo, hi


def _pk(lo, hi):
    lu = lax.bitcast_convert_type(lo, jnp.uint32)
    hu = lax.bitcast_convert_type(hi, jnp.uint32)
    w = ((lu + _RND) >> np.uint32(16)) | ((hu + _RND) & _MASKHI)
    return lax.bitcast_convert_type(w, jnp.int32)


def _segsum_body(tbl, a2b_flat, out, idx_v, rows_v, out_v,
                 sem0, sem1, sem2, sem3):
    wid = lax.axis_index("s") * NC + lax.axis_index("c")
    a0 = wid * A_PER_W
    R = AB * MAX_NB

    pltpu.sync_copy(a2b_flat.at[pl.ds(a0 * MAX_NB, A_PER_W * MAX_NB)], idx_v)

    sems = (sem0, sem1, sem2, sem3)

    def gather(t, u):
        return pltpu.async_copy(
            tbl.at[idx_v.at[pl.ds(t * R, R)]], rows_v.at[u], sems[u])

    for u in range(NBUF):
        gather(u, u)

    def outer(g, carry):
        for u in range(NBUF):
            t = g * NBUF + u
            pltpu.make_async_copy(
                tbl.at[idx_v.at[pl.ds(t * R, R)]], rows_v.at[u],
                sems[u]).wait()
            def atom_loop(a, c2, t=t, u=u):
                la = t * AB + a

                def red(rb, accs):
                    new = list(accs)
                    for dr in range(_RB):
                        r = rb * _RB + dr
                        for j in range(WCH):
                            lo, hi = _up(rows_v[u, a * MAX_NB + r,
                                                pl.ds(j * L, L)])
                            new[2 * j] = new[2 * j] + jnp.maximum(lo, 0.0)
                            new[2 * j + 1] = (new[2 * j + 1]
                                              + jnp.maximum(hi, 0.0))
                    return tuple(new)

                zeros = tuple(jnp.zeros((L,), jnp.float32)
                              for _ in range(2 * WCH))
                accs = lax.fori_loop(0, MAX_NB // _RB, red, zeros)
                for j in range(WCH):
                    out_v[la, pl.ds(j * L, L)] = _pk(accs[2 * j],
                                                     accs[2 * j + 1])
                return c2

            lax.fori_loop(0, AB, atom_loop, 0)

            @pl.when(g < SGN // NBUF - 1)
            def _():
                gather(t + NBUF, u)
        return carry

    lax.fori_loop(0, SGN // NBUF, outer, 0)
    pltpu.sync_copy(out_v, out.at[pl.ds(a0, A_PER_W)])


def _edge_body(amsg, s_tbl, b2a, b2revb, out, ia_v, ir_v, ga_v, gr_v, po_v,
               sem0, sem1, sem2, sem3, osem0, osem1, osem2, osem3):
    wid = lax.axis_index("s") * NC + lax.axis_index("c")
    b0 = wid * B_PER_W

    pltpu.sync_copy(b2a.at[pl.ds(b0, B_PER_W)], ia_v)
    pltpu.sync_copy(b2revb.at[pl.ds(b0, B_PER_W)], ir_v)

    sems = (sem0, sem1, sem2, sem3)
    osems = (osem0, osem1, osem2, osem3)

    def gather(t, u):
        pltpu.async_copy(amsg.at[ia_v.at[pl.ds(t * KB, KB)]], ga_v.at[u],
                         sems[u])
        pltpu.async_copy(s_tbl.at[ir_v.at[pl.ds(t * KB, KB)]], gr_v.at[u],
                         sems[u])

    def outcopy(t, u):
        return pltpu.make_async_copy(
            po_v.at[u], out.at[pl.ds(b0 + t * KB, KB)], osems[u])

    for u in range(NBUF):
        gather(u, u)

    def outer(g, carry):
        for u in range(NBUF):
            t = g * NBUF + u
            pltpu.make_async_copy(
                amsg.at[ia_v.at[pl.ds(t * KB, KB)]], ga_v.at[u],
                sems[u]).wait()
            pltpu.make_async_copy(
                s_tbl.at[ir_v.at[pl.ds(t * KB, KB)]], gr_v.at[u],
                sems[u]).wait()

            @pl.when(g >= 1)
            def _():
                outcopy(t - NBUF, u).wait()

            def rowblk(rb, c2):
                for dr in range(_RB):
                    r = rb * _RB + dr
                    for j in range(WCH):
                        sl = pl.ds(j * L, L)
                        alo, ahi = _up(ga_v[u, r, sl])
                        rlo, rhi = _up(gr_v[u, r, sl])
                        po_v[u, r, sl] = _pk(
                            alo - jnp.maximum(rlo, 0.0),
                            ahi - jnp.maximum(rhi, 0.0))
                return c2

            lax.fori_loop(0, KB // _RB, rowblk, 0)
            outcopy(t, u).start()

            @pl.when(g < EGN // NBUF - 1)
            def _():
                gather(t + NBUF, u)
        return carry

    lax.fori_loop(0, EGN // NBUF, outer, 0)
    for u in range(NBUF):
        outcopy(EGN - NBUF + u, u).wait()


_segsum_pk = pl.kernel(
    _segsum_body,
    out_type=jax.ShapeDtypeStruct((APAD, NWD), jnp.int32),
    mesh=_MESH,
    scratch_types=[
        pltpu.VMEM((A_PER_W * MAX_NB,), jnp.int32),
        pltpu.VMEM((NBUF, AB * MAX_NB, NWD), jnp.int32),
        pltpu.VMEM((A_PER_W, NWD), jnp.int32),
        pltpu.SemaphoreType.DMA,
        pltpu.SemaphoreType.DMA,
        pltpu.SemaphoreType.DMA,
        pltpu.SemaphoreType.DMA,
    ],
)

_edge = pl.kernel(
    _edge_body,
    out_type=jax.ShapeDtypeStruct((BPAD, NWD), jnp.int32),
    mesh=_MESH,
    scratch_types=[
        pltpu.VMEM((B_PER_W,), jnp.int32),
        pltpu.VMEM((B_PER_W,), jnp.int32),
        pltpu.VMEM((NBUF, KB, NWD), jnp.int32),
        pltpu.VMEM((NBUF, KB, NWD), jnp.int32),
        pltpu.VMEM((NBUF, KB, NWD), jnp.int32),
        pltpu.SemaphoreType.DMA,
        pltpu.SemaphoreType.DMA,
        pltpu.SemaphoreType.DMA,
        pltpu.SemaphoreType.DMA,
        pltpu.SemaphoreType.DMA,
        pltpu.SemaphoreType.DMA,
        pltpu.SemaphoreType.DMA,
        pltpu.SemaphoreType.DMA,
    ],
)

_TB = 512
_NTILES = N_BONDS // _TB


def _tc_pack(mm):
    lo = lax.bitcast_convert_type(mm[:, :NWD], jnp.uint32)
    hi = lax.bitcast_convert_type(mm[:, NWD:], jnp.uint32)
    w = ((lo + _RND) >> np.uint32(16)) | ((hi + _RND) & _MASKHI)
    return lax.bitcast_convert_type(w, jnp.int32)


def _tc_unpack(pw):
    u = lax.bitcast_convert_type(pw, jnp.uint32)
    lo = lax.bitcast_convert_type(u << np.uint32(16), jnp.float32)
    hi = lax.bitcast_convert_type(u & _MASKHI, jnp.float32)
    return jnp.concatenate([lo, hi], axis=1)


def _mm_body(x_ref, w_ref, o_ref):
    mm = jnp.dot(x_ref[...], w_ref[...], preferred_element_type=jnp.float32)
    o_ref[...] = _tc_pack(mm)


def _mm_skip_body(p_ref, i_ref, w_ref, o_ref):
    x = _tc_unpack(p_ref[...]).astype(jnp.bfloat16)
    mm = jnp.dot(x, w_ref[...], preferred_element_type=jnp.float32)
    o_ref[...] = _tc_pack(mm + _tc_unpack(i_ref[...]))


def _out_body(fa_ref, am_ref, wa_ref, wh_ref, b_ref, o_ref):
    acc = jnp.dot(fa_ref[...], wa_ref[...], preferred_element_type=jnp.float32)
    am = _tc_unpack(am_ref[...])
    acc = acc + jnp.dot(am, wh_ref[...], preferred_element_type=jnp.float32)
    o_ref[...] = jnp.maximum(acc + b_ref[...], 0.0)


def kernel(f_atoms, f_bonds, a2b, b2a, b2revb, W_i, W_h, W_o_w, W_o_b):
    a2b_flat = jnp.pad(a2b, ((0, APAD - N_ATOMS), (0, 0))).reshape(-1)
    b2a_p = jnp.pad(b2a, (0, BPAD - N_BONDS))
    b2revb_p = jnp.pad(b2revb, (0, BPAD - N_BONDS))

    W_i_q = W_i.T[:, Q]
    W_h_q = W_h.T[Q, :][:, Q].astype(jnp.bfloat16)

    s0 = pl.pallas_call(
        _mm_body,
        grid=(_NTILES,),
        in_specs=[
            pl.BlockSpec((_TB, BOND_FDIM), lambda i: (i, 0)),
            pl.BlockSpec((BOND_FDIM, HIDDEN), lambda i: (0, 0)),
        ],
        out_specs=pl.BlockSpec((_TB, NWD), lambda i: (i, 0)),
        out_shape=jax.ShapeDtypeStruct((BPAD, NWD), jnp.int32),
    )(f_bonds, W_i_q)

    s = s0
    for _ in range(DEPTH - 1):
        a_msg = _segsum_pk(s, a2b_flat)
        pre = _edge(a_msg, s, b2a_p, b2revb_p)
        s = pl.pallas_call(
            _mm_skip_body,
            grid=(_NTILES,),
            in_specs=[
                pl.BlockSpec((_TB, NWD), lambda i: (i, 0)),
                pl.BlockSpec((_TB, NWD), lambda i: (i, 0)),
                pl.BlockSpec((HIDDEN, HIDDEN), lambda i: (0, 0)),
            ],
            out_specs=pl.BlockSpec((_TB, NWD), lambda i: (i, 0)),
            out_shape=jax.ShapeDtypeStruct((BPAD, NWD), jnp.int32),
        )(pre, s0, W_h_q)

    a_sum_pk = _segsum_pk(s, a2b_flat)[:N_ATOMS]

    _TA = 400
    out = pl.pallas_call(
        _out_body,
        grid=(N_ATOMS // _TA,),
        in_specs=[
            pl.BlockSpec((_TA, ATOM_FDIM), lambda i: (i, 0)),
            pl.BlockSpec((_TA, NWD), lambda i: (i, 0)),
            pl.BlockSpec((ATOM_FDIM, HIDDEN), lambda i: (0, 0)),
            pl.BlockSpec((HIDDEN, HIDDEN), lambda i: (0, 0)),
            pl.BlockSpec((1, HIDDEN), lambda i: (0, 0)),
        ],
        out_specs=pl.BlockSpec((_TA, HIDDEN), lambda i: (i, 0)),
        out_shape=jax.ShapeDtypeStruct((N_ATOMS, HIDDEN), jnp.float32),
    )(f_atoms, a_sum_pk, W_o_w[:, :ATOM_FDIM].T,
      (W_o_w[:, ATOM_FDIM:].T / MAX_NB)[Q, :], W_o_b[None, :])

    return out.reshape(N_MOLS, N_ATOMS // N_MOLS, HIDDEN)


# speedup vs baseline: 1.1261x; 1.1261x over previous
import functools

import jax
import jax.numpy as jnp
import numpy as np
from jax import lax
from jax.experimental import pallas as pl
from jax.experimental.pallas import tpu as pltpu
from jax.experimental.pallas import tpu_sc as plsc

N_ATOMS = 10000
N_BONDS = 320000
MAX_NB = 32
ATOM_FDIM = 128
BOND_FDIM = 144
HIDDEN = 256
DEPTH = 3
N_MOLS = 100

NC, NS, L = 2, 16, 16
NW = NC * NS
NWD = HIDDEN // 2
WCH = NWD // L

_w = np.arange(NWD)
_LO = 32 * (_w // 16) + (_w % 16)
_HI = _LO + 16
Q = np.concatenate([_LO, _HI])

APAD = 10240
BPAD = 327680
A_PER_W = APAD // NW
B_PER_W = BPAD // NW
AB = 4
SGN = A_PER_W // AB
KB = 40
EGN = B_PER_W // KB
_RB = 8
NBUF = 4

_MESH = plsc.VectorSubcoreMesh(
    core_axis_name="c", subcore_axis_name="s", num_cores=NC, num_subcores=NS)

_MASKHI = np.uint32(0xFFFF0000)
_RND = np.uint32(0x8000)


def _up(w):
    u = lax.bitcast_convert_type(w, jnp.uint32)
    lo = lax.bitcast_convert_type(u << np.uint32(16), jnp.float32)
    hi = lax.bitcast_convert_type(u & _MASKHI, jnp.float32)
    return lo, hi


def _pk(lo, hi):
    lu = lax.bitcast_convert_type(lo, jnp.uint32)
    hu = lax.bitcast_convert_type(hi, jnp.uint32)
    w = ((lu + _RND) >> np.uint32(16)) | ((hu + _RND) & _MASKHI)
    return lax.bitcast_convert_type(w, jnp.int32)


def _segsum_body(tbl, a2b_flat, out, idx_v, rows_v, out_v,
                 sem0, sem1, sem2, sem3):
    wid = lax.axis_index("s") * NC + lax.axis_index("c")
    a0 = wid * A_PER_W
    R = AB * MAX_NB

    pltpu.sync_copy(a2b_flat.at[pl.ds(a0 * MAX_NB, A_PER_W * MAX_NB)], idx_v)

    sems = (sem0, sem1, sem2, sem3)

    def gather(t, u):
        return pltpu.async_copy(
            tbl.at[idx_v.at[pl.ds(t * R, R)]], rows_v.at[u], sems[u])

    for u in range(NBUF):
        gather(u, u)

    def outer(g, carry):
        for u in range(NBUF):
            t = g * NBUF + u
            pltpu.make_async_copy(
                tbl.at[idx_v.at[pl.ds(t * R, R)]], rows_v.at[u],
                sems[u]).wait()
            def atom_loop(a, c2, t=t, u=u):
                la = t * AB + a

                def red(rb, accs):
                    new = list(accs)
                    for dr in range(_RB):
                        r = rb * _RB + dr
                        for j in range(WCH):
                            lo, hi = _up(rows_v[u, a * MAX_NB + r,
                                                pl.ds(j * L, L)])
                            new[2 * j] = new[2 * j] + jnp.maximum(lo, 0.0)
                            new[2 * j + 1] = (new[2 * j + 1]
                                              + jnp.maximum(hi, 0.0))
                    return tuple(new)

                zeros = tuple(jnp.zeros((L,), jnp.float32)
                              for _ in range(2 * WCH))
                accs = lax.fori_loop(0, MAX_NB // _RB, red, zeros)
                for j in range(WCH):
                    out_v[la, pl.ds(j * L, L)] = _pk(accs[2 * j],
                                                     accs[2 * j + 1])
                return c2

            lax.fori_loop(0, AB, atom_loop, 0)

            @pl.when(g < SGN // NBUF - 1)
            def _():
                gather(t + NBUF, u)
        return carry

    lax.fori_loop(0, SGN // NBUF, outer, 0)
    pltpu.sync_copy(out_v, out.at[pl.ds(a0, A_PER_W)])


def _edge2_body(ah, h_tbl, inp0, b2a, b2revb, out, ia_v, ir_v,
                ga_v, gh_v, gi_v, po_v,
                sem0, sem1, sem2, sem3, osem0, osem1, osem2, osem3):
    wid = lax.axis_index("s") * NC + lax.axis_index("c")
    b0 = wid * B_PER_W

    pltpu.sync_copy(b2a.at[pl.ds(b0, B_PER_W)], ia_v)
    pltpu.sync_copy(b2revb.at[pl.ds(b0, B_PER_W)], ir_v)

    sems = (sem0, sem1, sem2, sem3)
    osems = (osem0, osem1, osem2, osem3)

    def gather(t, u):
        pltpu.async_copy(ah.at[ia_v.at[pl.ds(t * KB, KB)]], ga_v.at[u],
                         sems[u])
        pltpu.async_copy(h_tbl.at[ir_v.at[pl.ds(t * KB, KB)]], gh_v.at[u],
                         sems[u])
        pltpu.async_copy(inp0.at[pl.ds(b0 + t * KB, KB)], gi_v.at[u],
                         sems[u])

    def waitg(t, u):
        pltpu.make_async_copy(
            ah.at[ia_v.at[pl.ds(t * KB, KB)]], ga_v.at[u], sems[u]).wait()
        pltpu.make_async_copy(
            h_tbl.at[ir_v.at[pl.ds(t * KB, KB)]], gh_v.at[u], sems[u]).wait()
        pltpu.make_async_copy(
            inp0.at[pl.ds(b0 + t * KB, KB)], gi_v.at[u], sems[u]).wait()

    def outcopy(t, u):
        return pltpu.make_async_copy(
            po_v.at[u], out.at[pl.ds(b0 + t * KB, KB)], osems[u])

    for u in range(NBUF):
        gather(u, u)

    def outer(g, carry):
        for u in range(NBUF):
            t = g * NBUF + u
            waitg(t, u)

            @pl.when(g >= 1)
            def _():
                outcopy(t - NBUF, u).wait()

            def rowblk(rb, c2):
                for dr in range(_RB):
                    r = rb * _RB + dr
                    for j in range(WCH):
                        sl = pl.ds(j * L, L)
                        alo, ahi = _up(ga_v[u, r, sl])
                        hlo, hhi = _up(gh_v[u, r, sl])
                        ilo, ihi = _up(gi_v[u, r, sl])
                        po_v[u, r, sl] = _pk(ilo + alo - hlo,
                                             ihi + ahi - hhi)
                return c2

            lax.fori_loop(0, KB // _RB, rowblk, 0)
            outcopy(t, u).start()

            @pl.when(g < EGN // NBUF - 1)
            def _():
                gather(t + NBUF, u)
        return carry

    lax.fori_loop(0, EGN // NBUF, outer, 0)
    for u in range(NBUF):
        outcopy(EGN - NBUF + u, u).wait()


_segsum_pk = pl.kernel(
    _segsum_body,
    out_type=jax.ShapeDtypeStruct((APAD, NWD), jnp.int32),
    mesh=_MESH,
    scratch_types=[
        pltpu.VMEM((A_PER_W * MAX_NB,), jnp.int32),
        pltpu.VMEM((NBUF, AB * MAX_NB, NWD), jnp.int32),
        pltpu.VMEM((A_PER_W, NWD), jnp.int32),
        pltpu.SemaphoreType.DMA,
        pltpu.SemaphoreType.DMA,
        pltpu.SemaphoreType.DMA,
        pltpu.SemaphoreType.DMA,
    ],
)

_edge2 = pl.kernel(
    _edge2_body,
    out_type=jax.ShapeDtypeStruct((BPAD, NWD), jnp.int32),
    mesh=_MESH,
    scratch_types=[
        pltpu.VMEM((B_PER_W,), jnp.int32),
        pltpu.VMEM((B_PER_W,), jnp.int32),
        pltpu.VMEM((NBUF, KB, NWD), jnp.int32),
        pltpu.VMEM((NBUF, KB, NWD), jnp.int32),
        pltpu.VMEM((NBUF, KB, NWD), jnp.int32),
        pltpu.VMEM((NBUF, KB, NWD), jnp.int32),
        pltpu.SemaphoreType.DMA,
        pltpu.SemaphoreType.DMA,
        pltpu.SemaphoreType.DMA,
        pltpu.SemaphoreType.DMA,
        pltpu.SemaphoreType.DMA,
        pltpu.SemaphoreType.DMA,
        pltpu.SemaphoreType.DMA,
        pltpu.SemaphoreType.DMA,
    ],
)

_TB = 512
_NTILES = N_BONDS // _TB


def _tc_pack(mm):
    lo = lax.bitcast_convert_type(mm[:, :NWD], jnp.uint32)
    hi = lax.bitcast_convert_type(mm[:, NWD:], jnp.uint32)
    w = ((lo + _RND) >> np.uint32(16)) | ((hi + _RND) & _MASKHI)
    return lax.bitcast_convert_type(w, jnp.int32)


def _tc_unpack(pw):
    u = lax.bitcast_convert_type(pw, jnp.uint32)
    lo = lax.bitcast_convert_type(u << np.uint32(16), jnp.float32)
    hi = lax.bitcast_convert_type(u & _MASKHI, jnp.float32)
    return jnp.concatenate([lo, hi], axis=1)


def _mm_body(x_ref, w_ref, o_ref):
    mm = jnp.dot(x_ref[...], w_ref[...], preferred_element_type=jnp.float32)
    o_ref[...] = _tc_pack(mm)


def _mm_h_body(p_ref, w_ref, o_ref, *, relu_in):
    x = _tc_unpack(p_ref[...])
    if relu_in:
        x = jnp.maximum(x, 0.0)
    mm = jnp.dot(x.astype(jnp.bfloat16), w_ref[...],
                 preferred_element_type=jnp.float32)
    o_ref[...] = _tc_pack(mm)


def _out_body(fa_ref, am_ref, wa_ref, wh_ref, b_ref, o_ref):
    acc = jnp.dot(fa_ref[...], wa_ref[...], preferred_element_type=jnp.float32)
    am = _tc_unpack(am_ref[...])
    acc = acc + jnp.dot(am, wh_ref[...], preferred_element_type=jnp.float32)
    o_ref[...] = jnp.maximum(acc + b_ref[...], 0.0)


def kernel(f_atoms, f_bonds, a2b, b2a, b2revb, W_i, W_h, W_o_w, W_o_b):
    a2b_flat = jnp.pad(a2b, ((0, APAD - N_ATOMS), (0, 0))).reshape(-1)
    b2a_p = jnp.pad(b2a, (0, BPAD - N_BONDS))
    b2revb_p = jnp.pad(b2revb, (0, BPAD - N_BONDS))

    W_i_q = W_i.T[:, Q]
    W_h_q = W_h.T[Q, :][:, Q].astype(jnp.bfloat16)

    s0 = pl.pallas_call(
        _mm_body,
        grid=(_NTILES,),
        in_specs=[
            pl.BlockSpec((_TB, BOND_FDIM), lambda i: (i, 0)),
            pl.BlockSpec((BOND_FDIM, HIDDEN), lambda i: (0, 0)),
        ],
        out_specs=pl.BlockSpec((_TB, NWD), lambda i: (i, 0)),
        out_shape=jax.ShapeDtypeStruct((BPAD, NWD), jnp.int32),
    )(f_bonds, W_i_q)

    def mm_h(tbl, nrows, relu_in):
        return pl.pallas_call(
            functools.partial(_mm_h_body, relu_in=relu_in),
            grid=(nrows // _TB,),
            in_specs=[
                pl.BlockSpec((_TB, NWD), lambda i: (i, 0)),
                pl.BlockSpec((HIDDEN, HIDDEN), lambda i: (0, 0)),
            ],
            out_specs=pl.BlockSpec((_TB, NWD), lambda i: (i, 0)),
            out_shape=jax.ShapeDtypeStruct((tbl.shape[0], NWD), jnp.int32),
        )(tbl, W_h_q)

    s = s0
    for _ in range(DEPTH - 1):
        a_msg = _segsum_pk(s, a2b_flat)
        h = mm_h(s, N_BONDS, True)
        ah = mm_h(a_msg, APAD, False)
        s = _edge2(ah, h, s0, b2a_p, b2revb_p)

    a_sum_pk = _segsum_pk(s, a2b_flat)[:N_ATOMS]

    _TA = 400
    out = pl.pallas_call(
        _out_body,
        grid=(N_ATOMS // _TA,),
        in_specs=[
            pl.BlockSpec((_TA, ATOM_FDIM), lambda i: (i, 0)),
            pl.BlockSpec((_TA, NWD), lambda i: (i, 0)),
            pl.BlockSpec((ATOM_FDIM, HIDDEN), lambda i: (0, 0)),
            pl.BlockSpec((HIDDEN, HIDDEN), lambda i: (0, 0)),
            pl.BlockSpec((1, HIDDEN), lambda i: (0, 0)),
        ],
        out_specs=pl.BlockSpec((_TA, HIDDEN), lambda i: (i, 0)),
        out_shape=jax.ShapeDtypeStruct((N_ATOMS, HIDDEN), jnp.float32),
    )(f_atoms, a_sum_pk, W_o_w[:, :ATOM_FDIM].T,
      (W_o_w[:, ATOM_FDIM:].T / MAX_NB)[Q, :], W_o_b[None, :])

    return out.reshape(N_MOLS, N_ATOMS // N_MOLS, HIDDEN)
